# baseline (speedup 1.0000x reference)
import functools
import math

import jax
import jax.numpy as jnp
from jax import lax
from jax.experimental import pallas as pl
from jax.experimental.pallas import tpu as pltpu
from jax.experimental.pallas import tpu_sc as plsc

VOCAB = 30522
EMBED = 512
SCALE = math.sqrt(EMBED)


@functools.lru_cache(maxsize=None)
def _make_gather(B, D, C=40, NBUF=6, LEAD=4):
    info = plsc.get_sparse_core_info()
    NC, NS = info.num_cores, info.num_subcores
    NW = NC * NS
    assert B % NW == 0
    b_per_w = B // NW
    assert b_per_w % C == 0 and (C % 8) == 0
    nchunks = b_per_w // C
    hstart = max(0, NBUF - LEAD)
    nlaps = max(0, (nchunks - LEAD - hstart) // NBUF)
    tstart = hstart + nlaps * NBUF
    mesh = plsc.VectorSubcoreMesh(core_axis_name="c", subcore_axis_name="s")

    @functools.partial(
        pl.kernel,
        mesh=mesh,
        out_type=jax.ShapeDtypeStruct((B, D), jnp.float32),
        scratch_types=[
            pltpu.VMEM((b_per_w,), jnp.int32),
            pltpu.VMEM((NBUF, C, D), jnp.float32),
        ]
        + [pltpu.SemaphoreType.DMA] * (2 * NBUF),
    )
    def gather(table_hbm, idx_hbm, out_hbm, idx_v, bufs, *sems):
        gs, ws = sems[:NBUF], sems[NBUF:]
        wid = lax.axis_index("s") * NC + lax.axis_index("c")
        base = wid * b_per_w
        pltpu.sync_copy(idx_hbm.at[pl.ds(base, b_per_w)], idx_v)

        def _g(c, s):
            off = pl.multiple_of(c * C, 8)
            return pltpu.make_async_copy(
                table_hbm.at[idx_v.at[pl.ds(off, C)]], bufs.at[s], gs[s]
            )

        def _w(c, s):
            off = pl.multiple_of(c * C, 8)
            return pltpu.make_async_copy(
                bufs.at[s], out_hbm.at[pl.ds(base + off, C)], ws[s]
            )

        def _scale_chunk(s):
            def srow(r, carry):
                row = bufs.at[s].at[r]
                for j in range(D // 16):
                    sl = pl.ds(j * 16, 16)
                    row[sl] = row[sl] * SCALE
                return carry

            lax.fori_loop(0, C, srow, 0)

        def step(c, s, wait_w, issue_g):
            _g(c, s).wait()
            if issue_g:
                s2 = (s + LEAD) % NBUF
                if wait_w:
                    _w(c + LEAD - NBUF, s2).wait()
                _g(c + LEAD, s2).start()
            _scale_chunk(s)
            _w(c, s).start()

        for c in range(LEAD):
            _g(c, c % NBUF).start()

        for c in range(hstart):
            step(c, c % NBUF, wait_w=(c + LEAD >= NBUF),
                 issue_g=(c + LEAD < nchunks))

        def lap(k, carry):
            c0 = hstart + k * NBUF
            for s in range(NBUF):
                step(c0 + s, (hstart + s) % NBUF, wait_w=True, issue_g=True)
            return carry

        lax.fori_loop(0, nlaps, lap, 0)

        for c in range(tstart, nchunks):
            step(c, c % NBUF, wait_w=True, issue_g=(c + LEAD < nchunks))

        for c in range(nchunks - NBUF, nchunks):
            _w(c, c % NBUF).wait()

    return gather


def kernel(x, table):
    flat_idx = x.reshape(-1)
    out = _make_gather(flat_idx.shape[0], EMBED)(table, flat_idx)
    return out.reshape(x.shape + (EMBED,))


# speedup vs baseline: 1.0013x; 1.0013x over previous
import functools
import math

import jax
import jax.numpy as jnp
from jax import lax
from jax.experimental import pallas as pl
from jax.experimental.pallas import tpu as pltpu
from jax.experimental.pallas import tpu_sc as plsc

VOCAB = 30522
EMBED = 512
SCALE = math.sqrt(EMBED)


@functools.lru_cache(maxsize=None)
def _make_gather(B, D, C=32, NBUF=7, LEAD=5):
    info = plsc.get_sparse_core_info()
    NC, NS = info.num_cores, info.num_subcores
    NW = NC * NS
    assert B % NW == 0
    b_per_w = B // NW
    assert b_per_w % C == 0 and (C % 8) == 0
    nchunks = b_per_w // C
    hstart = max(0, NBUF - LEAD)
    nlaps = max(0, (nchunks - LEAD - hstart) // NBUF)
    tstart = hstart + nlaps * NBUF
    mesh = plsc.VectorSubcoreMesh(core_axis_name="c", subcore_axis_name="s")

    @functools.partial(
        pl.kernel,
        mesh=mesh,
        out_type=jax.ShapeDtypeStruct((B, D), jnp.float32),
        scratch_types=[
            pltpu.VMEM((b_per_w,), jnp.int32),
            pltpu.VMEM((NBUF, C, D), jnp.float32),
        ]
        + [pltpu.SemaphoreType.DMA] * (2 * NBUF),
    )
    def gather(table_hbm, idx_hbm, out_hbm, idx_v, bufs, *sems):
        gs, ws = sems[:NBUF], sems[NBUF:]
        wid = lax.axis_index("s") * NC + lax.axis_index("c")
        base = wid * b_per_w
        pltpu.sync_copy(idx_hbm.at[pl.ds(base, b_per_w)], idx_v)

        def _g(c, s):
            off = pl.multiple_of(c * C, 8)
            return pltpu.make_async_copy(
                table_hbm.at[idx_v.at[pl.ds(off, C)]], bufs.at[s], gs[s]
            )

        def _w(c, s):
            off = pl.multiple_of(c * C, 8)
            return pltpu.make_async_copy(
                bufs.at[s], out_hbm.at[pl.ds(base + off, C)], ws[s]
            )

        def _scale_chunk(s):
            def srow(r, carry):
                row = bufs.at[s].at[r]
                for j in range(D // 16):
                    sl = pl.ds(j * 16, 16)
                    row[sl] = row[sl] * SCALE
                return carry

            lax.fori_loop(0, C, srow, 0)

        def step(c, s, wait_w, issue_g):
            _g(c, s).wait()
            if issue_g:
                s2 = (s + LEAD) % NBUF
                if wait_w:
                    _w(c + LEAD - NBUF, s2).wait()
                _g(c + LEAD, s2).start()
            _scale_chunk(s)
            _w(c, s).start()

        for c in range(LEAD):
            _g(c, c % NBUF).start()

        for c in range(hstart):
            step(c, c % NBUF, wait_w=(c + LEAD >= NBUF),
                 issue_g=(c + LEAD < nchunks))

        def lap(k, carry):
            c0 = hstart + k * NBUF
            for s in range(NBUF):
                step(c0 + s, (hstart + s) % NBUF, wait_w=True, issue_g=True)
            return carry

        lax.fori_loop(0, nlaps, lap, 0)

        for c in range(tstart, nchunks):
            step(c, c % NBUF, wait_w=True, issue_g=(c + LEAD < nchunks))

        for c in range(nchunks - NBUF, nchunks):
            _w(c, c % NBUF).wait()

    return gather


def kernel(x, table):
    flat_idx = x.reshape(-1)
    out = _make_gather(flat_idx.shape[0], EMBED)(table, flat_idx)
    return out.reshape(x.shape + (EMBED,))


# speedup vs baseline: 1.0031x; 1.0018x over previous
import functools
import math

import jax
import jax.numpy as jnp
from jax import lax
from jax.experimental import pallas as pl
from jax.experimental.pallas import tpu as pltpu
from jax.experimental.pallas import tpu_sc as plsc

VOCAB = 30522
EMBED = 512
SCALE = math.sqrt(EMBED)


@functools.lru_cache(maxsize=None)
def _make_gather(B, D, C=32, NBUF=7, LEAD=5):
    info = plsc.get_sparse_core_info()
    NC, NS = info.num_cores, info.num_subcores
    NW = NC * NS
    assert B % NW == 0
    b_per_w = B // NW
    assert b_per_w % C == 0 and (C % 8) == 0
    nchunks = b_per_w // C
    hstart = max(0, NBUF - LEAD)
    nlaps = max(0, (nchunks - LEAD - hstart) // NBUF)
    tstart = hstart + nlaps * NBUF
    mesh = plsc.VectorSubcoreMesh(core_axis_name="c", subcore_axis_name="s")

    @functools.partial(
        pl.kernel,
        mesh=mesh,
        out_type=jax.ShapeDtypeStruct((B, D), jnp.float32),
        scratch_types=[
            pltpu.VMEM((b_per_w,), jnp.int32),
            pltpu.VMEM((NBUF, C, D), jnp.float32),
        ]
        + [pltpu.SemaphoreType.DMA] * (2 * NBUF),
    )
    def gather(table_hbm, idx_hbm, out_hbm, idx_v, bufs, *sems):
        gs, ws = sems[:NBUF], sems[NBUF:]
        wid = lax.axis_index("s") * NC + lax.axis_index("c")
        base = wid * b_per_w
        pltpu.sync_copy(idx_hbm.at[pl.ds(base, b_per_w)], idx_v)

        def _g(c, s):
            off = pl.multiple_of(c * C, 8)
            return pltpu.make_async_copy(
                table_hbm.at[idx_v.at[pl.ds(off, C)]], bufs.at[s], gs[s]
            )

        def _w(c, s):
            off = pl.multiple_of(c * C, 8)
            return pltpu.make_async_copy(
                bufs.at[s], out_hbm.at[pl.ds(base + off, C)], ws[s]
            )

        def _scale_chunk(s):
            def srow(r, carry):
                row = bufs.at[s].at[r]
                for j in range(D // 16):
                    sl = pl.ds(j * 16, 16)
                    row[sl] = row[sl] * SCALE
                return carry

            lax.fori_loop(0, C, srow, 0)

        def step(c, s, wait_w, issue_g):
            _g(c, s).wait()
            if issue_g:
                s2 = (s + LEAD) % NBUF
                if wait_w:
                    _w(c + LEAD - NBUF, s2).wait()
                _g(c + LEAD, s2).start()
            _scale_chunk(s)
            _w(c, s).start()

        for c in range(LEAD):
            _g(c, c % NBUF).start()

        for c in range(hstart):
            step(c, c % NBUF, wait_w=(c + LEAD >= NBUF),
                 issue_g=(c + LEAD < nchunks))

        def lap(k, carry):
            c0 = hstart + k * NBUF
            for s in range(NBUF):
                step(c0 + s, (hstart + s) % NBUF, wait_w=True, issue_g=True)
            return carry

        lax.fori_loop(0, nlaps, lap, 0)

        for c in range(tstart, nchunks):
            step(c, c % NBUF, wait_w=True, issue_g=(c + LEAD < nchunks))

        for c in range(nchunks - NBUF, nchunks):
            _w(c, c % NBUF).wait()

    return gather


def kernel(x, table):
    flat_idx = x.reshape(-1).astype(jnp.int32)
    out = _make_gather(flat_idx.shape[0], EMBED)(table, flat_idx)
    return out.reshape(x.shape + (EMBED,))
